# Initial kernel scaffold; baseline (speedup 1.0000x reference)
import functools

import jax
import jax.numpy as jnp
from jax import lax
from jax.experimental import pallas as pl
from jax.experimental.pallas import tpu as pltpu
from jax.experimental.pallas import tpu_sc as plsc

N_NODES = 10000
N_EDGES = 320000
D_IN = 128
H1 = 64
H2 = 32
N_HEADS = 3

NUM_SC = 2
NUM_TILES = 16
DEG_PAD = 10240

EDGES_PER_TILE = N_EDGES // (NUM_SC * NUM_TILES)
CHUNK = 128
N_FULL_CHUNKS = EDGES_PER_TILE // CHUNK
TAIL = EDGES_PER_TILE - N_FULL_CHUNKS * CHUNK

ROWS_PER_TILE = N_NODES // NUM_TILES
ZROWS = 125

_mesh = plsc.VectorSubcoreMesh(core_axis_name="c", subcore_axis_name="s")


def _sc_degree(dst):
    @functools.partial(
        pl.kernel,
        out_type=jax.ShapeDtypeStruct((NUM_SC, DEG_PAD), jnp.float32),
        mesh=_mesh,
        scratch_types=[
            pltpu.VMEM_SHARED((DEG_PAD,), jnp.float32),
            pltpu.VMEM((640,), jnp.float32),
            pltpu.VMEM((CHUNK,), jnp.int32),
            pltpu.VMEM((CHUNK,), jnp.float32),
            pltpu.VMEM((TAIL,), jnp.int32),
            pltpu.VMEM((TAIL,), jnp.float32),
            pltpu.SemaphoreType.DMA,
        ],
    )
    def deg_kernel(dst_hbm, out_hbm, acc, zv, idx_v, ones_v, idx_t, ones_t, sem):
        cid = lax.axis_index("c")
        sid = lax.axis_index("s")

        @pl.loop(0, 640, step=16)
        def _(i):
            zv[pl.ds(i, 16)] = jnp.zeros((16,), jnp.float32)

        @pl.loop(0, CHUNK, step=16)
        def _(i):
            ones_v[pl.ds(i, 16)] = jnp.full((16,), 1.0, jnp.float32)

        idx_t_init = pl.ds(0, 16)
        ones_t[idx_t_init] = jnp.full((16,), 1.0, jnp.float32)

        pltpu.sync_copy(zv, acc.at[pl.ds(sid * 640, 640)])
        plsc.subcore_barrier()

        ebase = cid * (N_EDGES // NUM_SC) + sid * EDGES_PER_TILE

        @pl.loop(0, N_FULL_CHUNKS * CHUNK, step=CHUNK)
        def _(c):
            pltpu.sync_copy(dst_hbm.at[pl.ds(ebase + c, CHUNK)], idx_v)
            pltpu.sync_copy(ones_v, acc.at[idx_v], add=True)

        pltpu.sync_copy(dst_hbm.at[pl.ds(ebase + N_FULL_CHUNKS * CHUNK, TAIL)],
                        idx_t)
        pltpu.sync_copy(ones_t, acc.at[idx_t], add=True)

        plsc.subcore_barrier()
        pltpu.sync_copy(acc.at[pl.ds(sid * 640, 640)],
                        out_hbm.at[cid, pl.ds(sid * 640, 640)])

    return deg_kernel(dst)


def _sc_aggregate(xs, src, dst, width):
    @functools.partial(
        pl.kernel,
        out_type=jax.ShapeDtypeStruct((NUM_SC, N_NODES, width), jnp.float32),
        mesh=_mesh,
        scratch_types=[
            pltpu.VMEM_SHARED((N_NODES, width), jnp.float32),
            pltpu.VMEM((ZROWS, width), jnp.float32),
            pltpu.VMEM((CHUNK,), jnp.int32),
            pltpu.VMEM((CHUNK,), jnp.int32),
            pltpu.VMEM((CHUNK, width), jnp.float32),
            pltpu.VMEM((TAIL,), jnp.int32),
            pltpu.VMEM((TAIL,), jnp.int32),
            pltpu.VMEM((TAIL, width), jnp.float32),
            pltpu.SemaphoreType.DMA,
        ],
    )
    def agg_kernel(xs_hbm, src_hbm, dst_hbm, out_hbm, acc, zbuf,
                   src_v, dst_v, rows_v, src_t, dst_t, rows_t, sem):
        cid = lax.axis_index("c")
        sid = lax.axis_index("s")

        @pl.loop(0, ZROWS)
        def _(i):
            @pl.loop(0, width, step=16)
            def _(j):
                zbuf[i, pl.ds(j, 16)] = jnp.zeros((16,), jnp.float32)

        rbase = sid * ROWS_PER_TILE

        @pl.loop(0, ROWS_PER_TILE, step=ZROWS)
        def _(r):
            pltpu.sync_copy(zbuf, acc.at[pl.ds(rbase + r, ZROWS)])

        plsc.subcore_barrier()

        ebase = cid * (N_EDGES // NUM_SC) + sid * EDGES_PER_TILE

        @pl.loop(0, N_FULL_CHUNKS * CHUNK, step=CHUNK)
        def _(c):
            pltpu.sync_copy(src_hbm.at[pl.ds(ebase + c, CHUNK)], src_v)
            pltpu.sync_copy(dst_hbm.at[pl.ds(ebase + c, CHUNK)], dst_v)
            pltpu.async_copy(xs_hbm.at[src_v], rows_v, sem).wait()
            pltpu.sync_copy(rows_v, acc.at[dst_v], add=True)

        toff = ebase + N_FULL_CHUNKS * CHUNK
        pltpu.sync_copy(src_hbm.at[pl.ds(toff, TAIL)], src_t)
        pltpu.sync_copy(dst_hbm.at[pl.ds(toff, TAIL)], dst_t)
        pltpu.async_copy(xs_hbm.at[src_t], rows_t, sem).wait()
        pltpu.sync_copy(rows_t, acc.at[dst_t], add=True)

        plsc.subcore_barrier()
        pltpu.sync_copy(acc.at[pl.ds(rbase, ROWS_PER_TILE)],
                        out_hbm.at[cid, pl.ds(rbase, ROWS_PER_TILE)])

    return agg_kernel(xs, src, dst)


_R_PREP = 2000
_R_DEC = 250


def _tc_prep(d0, d1, x):
    def body(d0_ref, d1_ref, x_ref, xs_ref, dinv_ref):
        deg = d0_ref[...] + d1_ref[...] + 1.0
        dinv = lax.rsqrt(deg)
        dinv_ref[...] = dinv
        xs_ref[...] = x_ref[...] * dinv

    grid = (N_NODES // _R_PREP,)
    return pl.pallas_call(
        body,
        grid=grid,
        in_specs=[
            pl.BlockSpec((_R_PREP, 1), lambda i: (i, 0)),
            pl.BlockSpec((_R_PREP, 1), lambda i: (i, 0)),
            pl.BlockSpec((_R_PREP, D_IN), lambda i: (i, 0)),
        ],
        out_specs=[
            pl.BlockSpec((_R_PREP, D_IN), lambda i: (i, 0)),
            pl.BlockSpec((_R_PREP, 1), lambda i: (i, 0)),
        ],
        out_shape=[
            jax.ShapeDtypeStruct((N_NODES, D_IN), jnp.float32),
            jax.ShapeDtypeStruct((N_NODES, 1), jnp.float32),
        ],
    )(d0, d1, x)


def _tc_hidden(p0, p1, xs, dinv, w_cat, b_cat):
    hw = N_HEADS * H1

    def body(p0_ref, p1_ref, xs_ref, dinv_ref, w_ref, b_ref, hs_ref):
        dinv = dinv_ref[...]
        agg = (p0_ref[...] + p1_ref[...] + xs_ref[...]) * dinv
        h = jnp.dot(agg, w_ref[...], preferred_element_type=jnp.float32)
        h = jnp.maximum(h + b_ref[...], 0.0)
        hs_ref[...] = h * dinv

    grid = (N_NODES // _R_PREP,)
    return pl.pallas_call(
        body,
        grid=grid,
        in_specs=[
            pl.BlockSpec((_R_PREP, D_IN), lambda i: (i, 0)),
            pl.BlockSpec((_R_PREP, D_IN), lambda i: (i, 0)),
            pl.BlockSpec((_R_PREP, D_IN), lambda i: (i, 0)),
            pl.BlockSpec((_R_PREP, 1), lambda i: (i, 0)),
            pl.BlockSpec((D_IN, hw), lambda i: (0, 0)),
            pl.BlockSpec((1, hw), lambda i: (0, 0)),
        ],
        out_specs=pl.BlockSpec((_R_PREP, hw), lambda i: (i, 0)),
        out_shape=jax.ShapeDtypeStruct((N_NODES, hw), jnp.float32),
    )(p0, p1, xs, dinv, w_cat, b_cat)


def _tc_heads(q0, q1, hs, dinv, wmu, bmu, wlv, blv):
    hw = N_HEADS * H1

    def body(q0_ref, q1_ref, hs_ref, dinv_ref, wmu_ref, bmu_ref,
             wlv_ref, blv_ref, mu_ref, lv_ref):
        agg = (q0_ref[...] + q1_ref[...] + hs_ref[...]) * dinv_ref[...]
        mu = None
        lv = None
        for i in range(N_HEADS):
            mu_i = jnp.dot(agg, wmu_ref[i],
                           preferred_element_type=jnp.float32) + bmu_ref[i]
            lv_i = jnp.dot(agg, wlv_ref[i],
                           preferred_element_type=jnp.float32) + blv_ref[i]
            mu = mu_i if mu is None else jnp.maximum(mu, mu_i)
            lv = lv_i if lv is None else jnp.maximum(lv, lv_i)
        mu_ref[...] = mu
        lv_ref[...] = lv

    grid = (N_NODES // _R_PREP,)
    return pl.pallas_call(
        body,
        grid=grid,
        in_specs=[
            pl.BlockSpec((_R_PREP, hw), lambda i: (i, 0)),
            pl.BlockSpec((_R_PREP, hw), lambda i: (i, 0)),
            pl.BlockSpec((_R_PREP, hw), lambda i: (i, 0)),
            pl.BlockSpec((_R_PREP, 1), lambda i: (i, 0)),
            pl.BlockSpec((N_HEADS, hw, H2), lambda i: (0, 0, 0)),
            pl.BlockSpec((N_HEADS, 1, H2), lambda i: (0, 0, 0)),
            pl.BlockSpec((N_HEADS, hw, H2), lambda i: (0, 0, 0)),
            pl.BlockSpec((N_HEADS, 1, H2), lambda i: (0, 0, 0)),
        ],
        out_specs=[
            pl.BlockSpec((_R_PREP, H2), lambda i: (i, 0)),
            pl.BlockSpec((_R_PREP, H2), lambda i: (i, 0)),
        ],
        out_shape=[
            jax.ShapeDtypeStruct((N_NODES, H2), jnp.float32),
            jax.ShapeDtypeStruct((N_NODES, H2), jnp.float32),
        ],
    )(q0, q1, hs, dinv, wmu, bmu, wlv, blv)


def _tc_decoder(z):
    def body(za_ref, zf_ref, out_ref):
        logits = lax.dot_general(
            za_ref[...], zf_ref[...], (((1,), (1,)), ((), ())),
            preferred_element_type=jnp.float32)
        out_ref[...] = jax.nn.sigmoid(logits)

    grid = (N_NODES // _R_DEC,)
    return pl.pallas_call(
        body,
        grid=grid,
        in_specs=[
            pl.BlockSpec((_R_DEC, H2), lambda i: (i, 0)),
            pl.BlockSpec((N_NODES, H2), lambda i: (0, 0)),
        ],
        out_specs=pl.BlockSpec((_R_DEC, N_NODES), lambda i: (i, 0)),
        out_shape=jax.ShapeDtypeStruct((N_NODES, N_NODES), jnp.float32),
    )(z, z)


def kernel(x, edge_index, W_gc, b_gc, W_mu, b_mu, W_lv, b_lv):
    src = edge_index[0]
    dst = edge_index[1]

    w_cat = jnp.concatenate([W_gc[i] for i in range(N_HEADS)], axis=1)
    b_cat = b_gc.reshape(1, N_HEADS * H1)
    wmu = jnp.stack([jnp.pad(W_mu[i], ((H1 * i, H1 * (N_HEADS - 1 - i)), (0, 0)))
                     for i in range(N_HEADS)])
    wlv = jnp.stack([jnp.pad(W_lv[i], ((H1 * i, H1 * (N_HEADS - 1 - i)), (0, 0)))
                     for i in range(N_HEADS)])
    bmu = b_mu[:, None, :]
    blv = b_lv[:, None, :]

    deg_parts = _sc_degree(dst)
    d0 = deg_parts[0, :N_NODES, None]
    d1 = deg_parts[1, :N_NODES, None]

    xs, dinv = _tc_prep(d0, d1, x)

    p = _sc_aggregate(xs, src, dst, D_IN)
    hs = _tc_hidden(p[0], p[1], xs, dinv, w_cat, b_cat)

    q = _sc_aggregate(hs, src, dst, N_HEADS * H1)
    mu_max, lv_max = _tc_heads(q[0], q[1], hs, dinv, wmu, bmu, wlv, blv)

    adj = _tc_decoder(mu_max)
    return (adj, mu_max, lv_max)


# speedup vs baseline: 21.6410x; 21.6410x over previous
import functools

import jax
import jax.numpy as jnp
from jax import lax
from jax.experimental import pallas as pl
from jax.experimental.pallas import tpu as pltpu
from jax.experimental.pallas import tpu_sc as plsc

N_NODES = 10000
N_EDGES = 320000
D_IN = 128
H1 = 64
H2 = 32
N_HEADS = 3

NUM_SC = 2
NUM_TILES = 16
DEG_PAD = 10240

EDGES_PER_TILE = N_EDGES // (NUM_SC * NUM_TILES)
CHUNK = 128
N_FULL_CHUNKS = EDGES_PER_TILE // CHUNK
TAIL = EDGES_PER_TILE - N_FULL_CHUNKS * CHUNK

N_PAD = 10240
ROWS_PER_TILE = N_PAD // NUM_TILES
ZROWS = 128

_mesh = plsc.VectorSubcoreMesh(core_axis_name="c", subcore_axis_name="s")


def _sc_degree(dst):
    @functools.partial(
        pl.kernel,
        out_type=jax.ShapeDtypeStruct((NUM_SC, DEG_PAD), jnp.float32),
        mesh=_mesh,
        scratch_types=[
            pltpu.VMEM_SHARED((DEG_PAD,), jnp.float32),
            pltpu.VMEM((640,), jnp.float32),
            pltpu.VMEM((CHUNK,), jnp.int32),
            pltpu.VMEM((CHUNK,), jnp.float32),
            pltpu.VMEM((TAIL,), jnp.int32),
            pltpu.VMEM((TAIL,), jnp.float32),
            pltpu.SemaphoreType.DMA,
        ],
    )
    def deg_kernel(dst_hbm, out_hbm, acc, zv, idx_v, ones_v, idx_t, ones_t, sem):
        cid = lax.axis_index("c")
        sid = lax.axis_index("s")

        @pl.loop(0, 640, step=16)
        def _(i):
            zv[pl.ds(i, 16)] = jnp.zeros((16,), jnp.float32)

        @pl.loop(0, CHUNK, step=16)
        def _(i):
            ones_v[pl.ds(i, 16)] = jnp.full((16,), 1.0, jnp.float32)

        ones_t[pl.ds(0, 16)] = jnp.full((16,), 1.0, jnp.float32)

        pltpu.sync_copy(zv, acc.at[pl.ds(sid * 640, 640)])
        plsc.subcore_barrier()

        ebase = cid * (N_EDGES // NUM_SC) + sid * EDGES_PER_TILE

        @pl.loop(0, N_FULL_CHUNKS * CHUNK, step=CHUNK)
        def _(c):
            pltpu.sync_copy(dst_hbm.at[pl.ds(ebase + c, CHUNK)], idx_v)
            pltpu.sync_copy(ones_v, acc.at[idx_v], add=True)

        pltpu.sync_copy(dst_hbm.at[pl.ds(ebase + N_FULL_CHUNKS * CHUNK, TAIL)],
                        idx_t)
        pltpu.sync_copy(ones_t, acc.at[idx_t], add=True)

        plsc.subcore_barrier()
        pltpu.sync_copy(acc.at[pl.ds(sid * 640, 640)],
                        out_hbm.at[cid, pl.ds(sid * 640, 640)])

    return deg_kernel(dst)


def _sc_aggregate(xs, src, dst):
    width = D_IN

    @functools.partial(
        pl.kernel,
        out_type=jax.ShapeDtypeStruct((NUM_SC, N_PAD, width), jnp.float32),
        mesh=_mesh,
        scratch_types=[
            pltpu.VMEM_SHARED((N_PAD, width), jnp.float32),
            pltpu.VMEM((ZROWS, width), jnp.float32),
            pltpu.VMEM((CHUNK,), jnp.int32),
            pltpu.VMEM((CHUNK,), jnp.int32),
            pltpu.VMEM((CHUNK, width), jnp.float32),
            pltpu.VMEM((TAIL,), jnp.int32),
            pltpu.VMEM((TAIL,), jnp.int32),
            pltpu.VMEM((TAIL, width), jnp.float32),
            pltpu.SemaphoreType.DMA,
        ],
    )
    def agg_kernel(xs_hbm, src_hbm, dst_hbm, out_hbm, acc, zbuf,
                   src_v, dst_v, rows_v, src_t, dst_t, rows_t, sem):
        cid = lax.axis_index("c")
        sid = lax.axis_index("s")

        @pl.loop(0, ZROWS)
        def _(i):
            @pl.loop(0, width, step=16)
            def _(j):
                zbuf[i, pl.ds(j, 16)] = jnp.zeros((16,), jnp.float32)

        rbase = sid * ROWS_PER_TILE

        @pl.loop(0, ROWS_PER_TILE, step=ZROWS)
        def _(r):
            pltpu.sync_copy(zbuf, acc.at[pl.ds(rbase + r, ZROWS)])

        plsc.subcore_barrier()

        ebase = cid * (N_EDGES // NUM_SC) + sid * EDGES_PER_TILE

        @pl.loop(0, N_FULL_CHUNKS * CHUNK, step=CHUNK)
        def _(c):
            pltpu.sync_copy(src_hbm.at[pl.ds(ebase + c, CHUNK)], src_v)
            pltpu.sync_copy(dst_hbm.at[pl.ds(ebase + c, CHUNK)], dst_v)
            pltpu.async_copy(xs_hbm.at[src_v], rows_v, sem).wait()
            pltpu.sync_copy(rows_v, acc.at[dst_v], add=True)

        toff = ebase + N_FULL_CHUNKS * CHUNK
        pltpu.sync_copy(src_hbm.at[pl.ds(toff, TAIL)], src_t)
        pltpu.sync_copy(dst_hbm.at[pl.ds(toff, TAIL)], dst_t)
        pltpu.async_copy(xs_hbm.at[src_t], rows_t, sem).wait()
        pltpu.sync_copy(rows_t, acc.at[dst_t], add=True)

        plsc.subcore_barrier()
        pltpu.sync_copy(acc.at[pl.ds(rbase, ROWS_PER_TILE)],
                        out_hbm.at[cid, pl.ds(rbase, ROWS_PER_TILE)])

    return agg_kernel(xs, src, dst)


_R_PREP = 2000
_R_DEC = 400


def _tc_prep(d0, d1, x):
    def body(d0_ref, d1_ref, x_ref, xs_ref, dinv_ref):
        deg = d0_ref[...] + d1_ref[...] + 1.0
        dinv = lax.rsqrt(deg)
        dinv_ref[...] = dinv
        xs_ref[...] = x_ref[...] * dinv

    grid = (N_NODES // _R_PREP,)
    return pl.pallas_call(
        body,
        grid=grid,
        in_specs=[
            pl.BlockSpec((_R_PREP, 1), lambda i: (i, 0)),
            pl.BlockSpec((_R_PREP, 1), lambda i: (i, 0)),
            pl.BlockSpec((_R_PREP, D_IN), lambda i: (i, 0)),
        ],
        out_specs=[
            pl.BlockSpec((_R_PREP, D_IN), lambda i: (i, 0)),
            pl.BlockSpec((_R_PREP, 1), lambda i: (i, 0)),
        ],
        out_shape=[
            jax.ShapeDtypeStruct((N_NODES, D_IN), jnp.float32),
            jax.ShapeDtypeStruct((N_NODES, 1), jnp.float32),
        ],
    )(d0, d1, x)


def _tc_hidden(p0, p1, xs, dinv, w_a, b_a, w_b, b_b):
    def body(p0_ref, p1_ref, xs_ref, dinv_ref, wa_ref, ba_ref,
             wb_ref, bb_ref, ha_ref, hb_ref):
        dinv = dinv_ref[...]
        agg = (p0_ref[...] + p1_ref[...] + xs_ref[...]) * dinv
        ha = jnp.dot(agg, wa_ref[...], preferred_element_type=jnp.float32)
        ha = jnp.maximum(ha + ba_ref[...], 0.0)
        hb = jnp.dot(agg, wb_ref[...], preferred_element_type=jnp.float32)
        hb = jnp.maximum(hb + bb_ref[...], 0.0)
        ha_ref[...] = ha * dinv
        hb_ref[...] = hb * dinv

    grid = (N_NODES // _R_PREP,)
    rb = lambda i: (i, 0)
    return pl.pallas_call(
        body,
        grid=grid,
        in_specs=[
            pl.BlockSpec((_R_PREP, D_IN), rb),
            pl.BlockSpec((_R_PREP, D_IN), rb),
            pl.BlockSpec((_R_PREP, D_IN), rb),
            pl.BlockSpec((_R_PREP, 1), rb),
            pl.BlockSpec((D_IN, D_IN), lambda i: (0, 0)),
            pl.BlockSpec((1, D_IN), lambda i: (0, 0)),
            pl.BlockSpec((D_IN, D_IN), lambda i: (0, 0)),
            pl.BlockSpec((1, D_IN), lambda i: (0, 0)),
        ],
        out_specs=[
            pl.BlockSpec((_R_PREP, D_IN), rb),
            pl.BlockSpec((_R_PREP, D_IN), rb),
        ],
        out_shape=[
            jax.ShapeDtypeStruct((N_NODES, D_IN), jnp.float32),
            jax.ShapeDtypeStruct((N_NODES, D_IN), jnp.float32),
        ],
    )(p0, p1, xs, dinv, w_a, b_a, w_b, b_b)


def _tc_heads(qa0, qa1, ha, qb0, qb1, hb, dinv, wmu_a, wmu_b, bmu,
              wlv_a, wlv_b, blv):
    def body(qa0_ref, qa1_ref, ha_ref, qb0_ref, qb1_ref, hb_ref, dinv_ref,
             wmua_ref, wmub_ref, bmu_ref, wlva_ref, wlvb_ref, blv_ref,
             mu_ref, lv_ref):
        dinv = dinv_ref[...]
        agg_a = (qa0_ref[...] + qa1_ref[...] + ha_ref[...]) * dinv
        agg_b = (qb0_ref[...] + qb1_ref[...] + hb_ref[...]) * dinv
        mu = None
        lv = None
        for i in range(N_HEADS):
            mu_i = (jnp.dot(agg_a, wmua_ref[i],
                            preferred_element_type=jnp.float32)
                    + jnp.dot(agg_b, wmub_ref[i],
                              preferred_element_type=jnp.float32)
                    + bmu_ref[i])
            lv_i = (jnp.dot(agg_a, wlva_ref[i],
                            preferred_element_type=jnp.float32)
                    + jnp.dot(agg_b, wlvb_ref[i],
                              preferred_element_type=jnp.float32)
                    + blv_ref[i])
            mu = mu_i if mu is None else jnp.maximum(mu, mu_i)
            lv = lv_i if lv is None else jnp.maximum(lv, lv_i)
        mu_ref[...] = mu
        lv_ref[...] = lv

    grid = (N_NODES // _R_PREP,)
    rb = lambda i: (i, 0)
    full3 = lambda i: (0, 0, 0)
    return pl.pallas_call(
        body,
        grid=grid,
        in_specs=[
            pl.BlockSpec((_R_PREP, D_IN), rb),
            pl.BlockSpec((_R_PREP, D_IN), rb),
            pl.BlockSpec((_R_PREP, D_IN), rb),
            pl.BlockSpec((_R_PREP, D_IN), rb),
            pl.BlockSpec((_R_PREP, D_IN), rb),
            pl.BlockSpec((_R_PREP, D_IN), rb),
            pl.BlockSpec((_R_PREP, 1), rb),
            pl.BlockSpec((N_HEADS, D_IN, H2), full3),
            pl.BlockSpec((N_HEADS, D_IN, H2), full3),
            pl.BlockSpec((N_HEADS, 1, H2), full3),
            pl.BlockSpec((N_HEADS, D_IN, H2), full3),
            pl.BlockSpec((N_HEADS, D_IN, H2), full3),
            pl.BlockSpec((N_HEADS, 1, H2), full3),
        ],
        out_specs=[
            pl.BlockSpec((_R_PREP, H2), rb),
            pl.BlockSpec((_R_PREP, H2), rb),
        ],
        out_shape=[
            jax.ShapeDtypeStruct((N_NODES, H2), jnp.float32),
            jax.ShapeDtypeStruct((N_NODES, H2), jnp.float32),
        ],
    )(qa0, qa1, ha, qb0, qb1, hb, dinv, wmu_a, wmu_b, bmu, wlv_a, wlv_b, blv)


def _tc_decoder(z):
    def body(za_ref, zf_ref, out_ref):
        logits = lax.dot_general(
            za_ref[...], zf_ref[...], (((1,), (1,)), ((), ())),
            preferred_element_type=jnp.float32)
        out_ref[...] = jax.nn.sigmoid(logits)

    grid = (N_NODES // _R_DEC,)
    return pl.pallas_call(
        body,
        grid=grid,
        in_specs=[
            pl.BlockSpec((_R_DEC, H2), lambda i: (i, 0)),
            pl.BlockSpec((N_NODES, H2), lambda i: (0, 0)),
        ],
        out_specs=pl.BlockSpec((_R_DEC, N_NODES), lambda i: (i, 0)),
        out_shape=jax.ShapeDtypeStruct((N_NODES, N_NODES), jnp.float32),
    )(z, z)


def kernel(x, edge_index, W_gc, b_gc, W_mu, b_mu, W_lv, b_lv):
    src = edge_index[0]
    dst = edge_index[1]

    w_a = jnp.concatenate([W_gc[0], W_gc[1]], axis=1)
    b_a = jnp.concatenate([b_gc[0], b_gc[1]])[None, :]
    w_b = jnp.pad(W_gc[2], ((0, 0), (0, H1)))
    b_b = jnp.pad(b_gc[2], (0, H1))[None, :]
    zpad = jnp.zeros((H1, H2), jnp.float32)
    wmu_a = jnp.stack([jnp.concatenate([W_mu[0], zpad]),
                       jnp.concatenate([zpad, W_mu[1]]),
                       jnp.concatenate([zpad, zpad])])
    wmu_b = jnp.stack([jnp.concatenate([zpad, zpad]),
                       jnp.concatenate([zpad, zpad]),
                       jnp.concatenate([W_mu[2], zpad])])
    wlv_a = jnp.stack([jnp.concatenate([W_lv[0], zpad]),
                       jnp.concatenate([zpad, W_lv[1]]),
                       jnp.concatenate([zpad, zpad])])
    wlv_b = jnp.stack([jnp.concatenate([zpad, zpad]),
                       jnp.concatenate([zpad, zpad]),
                       jnp.concatenate([W_lv[2], zpad])])
    bmu = b_mu[:, None, :]
    blv = b_lv[:, None, :]

    deg_parts = _sc_degree(dst)
    d0 = deg_parts[0, :N_NODES, None]
    d1 = deg_parts[1, :N_NODES, None]

    xs, dinv = _tc_prep(d0, d1, x)

    p = _sc_aggregate(xs, src, dst)
    ha, hb = _tc_hidden(p[0, :N_NODES], p[1, :N_NODES], xs, dinv,
                        w_a, b_a, w_b, b_b)

    qa = _sc_aggregate(ha, src, dst)
    qb = _sc_aggregate(hb, src, dst)
    mu_max, lv_max = _tc_heads(qa[0, :N_NODES], qa[1, :N_NODES], ha,
                               qb[0, :N_NODES], qb[1, :N_NODES], hb,
                               dinv, wmu_a, wmu_b, bmu, wlv_a, wlv_b, blv)

    adj = _tc_decoder(mu_max)
    return (adj, mu_max, lv_max)
